# Initial kernel scaffold; baseline (speedup 1.0000x reference)
import jax
import jax.numpy as jnp
from jax.experimental import pallas as pl


def kernel(pred, target):
    raise NotImplementedError("write your pallas kernel here")


# speedup vs baseline: 10.1841x; 10.1841x over previous
import jax
import jax.numpy as jnp
from jax.experimental import pallas as pl
from jax.experimental.pallas import tpu as pltpu

_IGNORE = 255
_THRESH = 0.7
_K = 131072

_B, _C, _H, _W = 8, 19, 512, 512
_RB = 64
_NR = _H // _RB

_TOP_BITS = 0x3F800000


def _main_body(pred_ref, tgt_ref, p_ref, nl_ref, cnt_ref, sum_ref):
    x = pred_ref[0]
    t = tgt_ref[0]
    m = jnp.max(x, axis=0)
    xs = x - m[None, :, :]
    e = jnp.exp(xs)
    s = jnp.sum(e, axis=0)
    cls = jax.lax.broadcasted_iota(jnp.int32, (_C, _RB, _W), 0)
    onehot = cls == t[None, :, :]
    et = jnp.sum(jnp.where(onehot, e, 0.0), axis=0)
    shift = jnp.sum(jnp.where(onehot, xs, 0.0), axis=0)
    valid = t != _IGNORE
    p = et / s
    p_eff = jnp.where(valid, p, 1.0)
    nl = jnp.log(s) - shift
    snl = jnp.where(valid, nl, -1.0)
    p_ref[0] = p_eff
    nl_ref[0] = snl
    kept = p_eff <= _THRESH
    cpart = jnp.sum(kept.astype(jnp.float32), axis=0)
    spart = jnp.sum(jnp.where(kept, nl, 0.0), axis=0)
    first = (pl.program_id(0) == 0) & (pl.program_id(1) == 0)

    @pl.when(first)
    def _init():
        cnt_ref[...] = cpart[None, :]
        sum_ref[...] = spart[None, :]

    @pl.when(jnp.logical_not(first))
    def _acc():
        cnt_ref[...] = cnt_ref[...] + cpart[None, :]
        sum_ref[...] = sum_ref[...] + spart[None, :]


def _count_body(mid_ref, bits_ref, cnt_ref):
    mid = mid_ref[0]
    le = bits_ref[0] <= mid
    cpart = jnp.sum(le.astype(jnp.int32), axis=0)
    first = pl.program_id(0) == 0

    @pl.when(first)
    def _init():
        cnt_ref[...] = cpart[None, :]

    @pl.when(jnp.logical_not(first))
    def _acc():
        cnt_ref[...] = cnt_ref[...] + cpart[None, :]


def _final_body(thr_ref, p_ref, nl_ref, cnt_ref, sum_ref):
    thr = thr_ref[0]
    p = p_ref[0]
    v = nl_ref[0]
    kept = (p <= thr) & (v >= -0.5)
    cpart = jnp.sum(kept.astype(jnp.float32), axis=0)
    spart = jnp.sum(jnp.where(kept, v, 0.0), axis=0)
    first = pl.program_id(0) == 0

    @pl.when(first)
    def _init():
        cnt_ref[...] = cpart[None, :]
        sum_ref[...] = spart[None, :]

    @pl.when(jnp.logical_not(first))
    def _acc():
        cnt_ref[...] = cnt_ref[...] + cpart[None, :]
        sum_ref[...] = sum_ref[...] + spart[None, :]


_main_call = pl.pallas_call(
    _main_body,
    grid=(_B, _NR),
    in_specs=[
        pl.BlockSpec((1, _C, _RB, _W), lambda i, j: (i, 0, j, 0)),
        pl.BlockSpec((1, _RB, _W), lambda i, j: (i, j, 0)),
    ],
    out_specs=[
        pl.BlockSpec((1, _RB, _W), lambda i, j: (i, j, 0)),
        pl.BlockSpec((1, _RB, _W), lambda i, j: (i, j, 0)),
        pl.BlockSpec((1, _W), lambda i, j: (0, 0)),
        pl.BlockSpec((1, _W), lambda i, j: (0, 0)),
    ],
    out_shape=[
        jax.ShapeDtypeStruct((_B, _H, _W), jnp.float32),
        jax.ShapeDtypeStruct((_B, _H, _W), jnp.float32),
        jax.ShapeDtypeStruct((1, _W), jnp.float32),
        jax.ShapeDtypeStruct((1, _W), jnp.float32),
    ],
)

_count_call = pl.pallas_call(
    _count_body,
    grid=(_B,),
    in_specs=[
        pl.BlockSpec(memory_space=pltpu.SMEM),
        pl.BlockSpec((1, _H, _W), lambda i: (i, 0, 0)),
    ],
    out_specs=[pl.BlockSpec((1, _W), lambda i: (0, 0))],
    out_shape=[jax.ShapeDtypeStruct((1, _W), jnp.int32)],
)

_final_call = pl.pallas_call(
    _final_body,
    grid=(_B,),
    in_specs=[
        pl.BlockSpec(memory_space=pltpu.SMEM),
        pl.BlockSpec((1, _H, _W), lambda i: (i, 0, 0)),
        pl.BlockSpec((1, _H, _W), lambda i: (i, 0, 0)),
    ],
    out_specs=[
        pl.BlockSpec((1, _W), lambda i: (0, 0)),
        pl.BlockSpec((1, _W), lambda i: (0, 0)),
    ],
    out_shape=[
        jax.ShapeDtypeStruct((1, _W), jnp.float32),
        jax.ShapeDtypeStruct((1, _W), jnp.float32),
    ],
)


def kernel(pred, target):
    p_eff, snl, cl, sl = _main_call(pred, target)
    bits = jax.lax.bitcast_convert_type(p_eff, jnp.int32)

    def body(_, lohi):
        lo, hi = lohi
        mid = (lo + hi) // 2
        cnt = jnp.sum(_count_call(mid.reshape(1), bits)[0])
        take_low = cnt >= _K
        return (jnp.where(take_low, lo, mid + 1),
                jnp.where(take_low, mid, hi))

    lo, hi = jax.lax.fori_loop(
        0, 31, body, (jnp.int32(0), jnp.int32(_TOP_BITS)))
    kth = jax.lax.bitcast_convert_type(lo, jnp.float32)
    thr = jnp.maximum(kth, jnp.float32(_THRESH))

    cf, sf = _final_call(thr.reshape(1), p_eff, snl)
    cnt = jnp.sum(cf)
    tot = jnp.sum(sf)
    return tot / jnp.maximum(cnt, 1.0)


# speedup vs baseline: 40.2553x; 3.9527x over previous
import jax
import jax.numpy as jnp
from jax.experimental import pallas as pl
from jax.experimental.pallas import tpu as pltpu

_IGNORE = 255
_THRESH = 0.7
_K = 131072

_B, _C, _H, _W = 8, 19, 512, 512
_RB = 64
_NR = _H // _RB

_TOP_BITS = 0x3F800000


def _main_body(pred_ref, tgt_ref, p_ref, nl_ref, cnt_ref, sum_ref):
    x = pred_ref[0]
    t = tgt_ref[0]
    m = jnp.max(x, axis=0)
    xs = x - m[None, :, :]
    e = jnp.exp(xs)
    s = jnp.sum(e, axis=0)
    cls = jax.lax.broadcasted_iota(jnp.int32, (_C, _RB, _W), 0)
    onehot = cls == t[None, :, :]
    et = jnp.sum(jnp.where(onehot, e, 0.0), axis=0)
    shift = jnp.sum(jnp.where(onehot, xs, 0.0), axis=0)
    valid = t != _IGNORE
    p = et / s
    p_eff = jnp.where(valid, p, 1.0)
    nl = jnp.log(s) - shift
    snl = jnp.where(valid, nl, -1.0)
    p_ref[0] = p_eff
    nl_ref[0] = snl
    kept = p_eff <= _THRESH
    cpart = jnp.sum(kept.astype(jnp.float32), axis=0)
    spart = jnp.sum(jnp.where(kept, nl, 0.0), axis=0)
    first = (pl.program_id(0) == 0) & (pl.program_id(1) == 0)

    @pl.when(first)
    def _init():
        cnt_ref[...] = cpart[None, :]
        sum_ref[...] = spart[None, :]

    @pl.when(jnp.logical_not(first))
    def _acc():
        cnt_ref[...] = cnt_ref[...] + cpart[None, :]
        sum_ref[...] = sum_ref[...] + spart[None, :]


def _count_body(mid_ref, bits_ref, cnt_ref):
    mid = mid_ref[0]
    le = bits_ref[0] <= mid
    cpart = jnp.sum(le.astype(jnp.int32), axis=0)
    first = pl.program_id(0) == 0

    @pl.when(first)
    def _init():
        cnt_ref[...] = cpart[None, :]

    @pl.when(jnp.logical_not(first))
    def _acc():
        cnt_ref[...] = cnt_ref[...] + cpart[None, :]


def _final_body(thr_ref, p_ref, nl_ref, cnt_ref, sum_ref):
    thr = thr_ref[0]
    p = p_ref[0]
    v = nl_ref[0]
    kept = (p <= thr) & (v >= -0.5)
    cpart = jnp.sum(kept.astype(jnp.float32), axis=0)
    spart = jnp.sum(jnp.where(kept, v, 0.0), axis=0)
    first = pl.program_id(0) == 0

    @pl.when(first)
    def _init():
        cnt_ref[...] = cpart[None, :]
        sum_ref[...] = spart[None, :]

    @pl.when(jnp.logical_not(first))
    def _acc():
        cnt_ref[...] = cnt_ref[...] + cpart[None, :]
        sum_ref[...] = sum_ref[...] + spart[None, :]


_main_call = pl.pallas_call(
    _main_body,
    grid=(_B, _NR),
    in_specs=[
        pl.BlockSpec((1, _C, _RB, _W), lambda i, j: (i, 0, j, 0)),
        pl.BlockSpec((1, _RB, _W), lambda i, j: (i, j, 0)),
    ],
    out_specs=[
        pl.BlockSpec((1, _RB, _W), lambda i, j: (i, j, 0)),
        pl.BlockSpec((1, _RB, _W), lambda i, j: (i, j, 0)),
        pl.BlockSpec((1, _W), lambda i, j: (0, 0)),
        pl.BlockSpec((1, _W), lambda i, j: (0, 0)),
    ],
    out_shape=[
        jax.ShapeDtypeStruct((_B, _H, _W), jnp.float32),
        jax.ShapeDtypeStruct((_B, _H, _W), jnp.float32),
        jax.ShapeDtypeStruct((1, _W), jnp.float32),
        jax.ShapeDtypeStruct((1, _W), jnp.float32),
    ],
)

_count_call = pl.pallas_call(
    _count_body,
    grid=(_B,),
    in_specs=[
        pl.BlockSpec(memory_space=pltpu.SMEM),
        pl.BlockSpec((1, _H, _W), lambda i: (i, 0, 0)),
    ],
    out_specs=[pl.BlockSpec((1, _W), lambda i: (0, 0))],
    out_shape=[jax.ShapeDtypeStruct((1, _W), jnp.int32)],
)

_final_call = pl.pallas_call(
    _final_body,
    grid=(_B,),
    in_specs=[
        pl.BlockSpec(memory_space=pltpu.SMEM),
        pl.BlockSpec((1, _H, _W), lambda i: (i, 0, 0)),
        pl.BlockSpec((1, _H, _W), lambda i: (i, 0, 0)),
    ],
    out_specs=[
        pl.BlockSpec((1, _W), lambda i: (0, 0)),
        pl.BlockSpec((1, _W), lambda i: (0, 0)),
    ],
    out_shape=[
        jax.ShapeDtypeStruct((1, _W), jnp.float32),
        jax.ShapeDtypeStruct((1, _W), jnp.float32),
    ],
)


def kernel(pred, target):
    p_eff, snl, cl, sl = _main_call(pred, target)
    cnt07 = jnp.sum(cl)
    sum07 = jnp.sum(sl)

    def common():
        return sum07 / jnp.maximum(cnt07, 1.0)

    def rare():
        bits = jax.lax.bitcast_convert_type(p_eff, jnp.int32)

        def body(_, lohi):
            lo, hi = lohi
            mid = (lo + hi) // 2
            cnt = jnp.sum(_count_call(mid.reshape(1), bits)[0])
            take_low = cnt >= _K
            return (jnp.where(take_low, lo, mid + 1),
                    jnp.where(take_low, mid, hi))

        lo, _ = jax.lax.fori_loop(
            0, 31, body, (jnp.int32(0), jnp.int32(_TOP_BITS)))
        kth = jax.lax.bitcast_convert_type(lo, jnp.float32)
        thr = jnp.maximum(kth, jnp.float32(_THRESH))

        cf, sf = _final_call(thr.reshape(1), p_eff, snl)
        return jnp.sum(sf) / jnp.maximum(jnp.sum(cf), 1.0)

    return jax.lax.cond(cnt07 >= _K, common, rare)


# speedup vs baseline: 41.4137x; 1.0288x over previous
import functools

import jax
import jax.numpy as jnp
from jax.experimental import pallas as pl
from jax.experimental.pallas import tpu as pltpu
from jax.experimental.pallas import tpu_sc as plsc

_IGNORE = 255
_THRESH = 0.7
_K = 131072

_B, _C, _H, _W = 8, 19, 512, 512
_RB = 64
_NR = _H // _RB

_TOP_BITS = 0x3F800000


def _pixel_stats(pred_ref, tgt_ref):
    x = pred_ref[0]
    t = tgt_ref[0]
    m = jnp.max(x, axis=0)
    xs = x - m[None, :, :]
    e = jnp.exp(xs)
    s = jnp.sum(e, axis=0)
    cls = jax.lax.broadcasted_iota(jnp.int32, (_C, _RB, _W), 0)
    onehot = cls == t[None, :, :]
    shift = jnp.sum(jnp.where(onehot, xs, 0.0), axis=0)
    valid = t != _IGNORE
    p = jnp.exp(shift) / s
    p_eff = jnp.where(valid, p, 1.0)
    nl = jnp.log(s) - shift
    snl = jnp.where(valid, nl, -1.0)
    return p_eff, snl, nl


def _main_body(pred_ref, tgt_ref, cnt_ref, sum_ref):
    p_eff, _, nl = _pixel_stats(pred_ref, tgt_ref)
    kept = p_eff <= _THRESH
    cpart = jnp.sum(kept.astype(jnp.float32), axis=0)
    spart = jnp.sum(jnp.where(kept, nl, 0.0), axis=0)
    first = (pl.program_id(0) == 0) & (pl.program_id(1) == 0)

    @pl.when(first)
    def _init():
        cnt_ref[...] = cpart[None, :]
        sum_ref[...] = spart[None, :]

    @pl.when(jnp.logical_not(first))
    def _acc():
        cnt_ref[...] = cnt_ref[...] + cpart[None, :]
        sum_ref[...] = sum_ref[...] + spart[None, :]


def _mat_body(pred_ref, tgt_ref, p_ref, nl_ref):
    p_eff, snl, _ = _pixel_stats(pred_ref, tgt_ref)
    p_ref[0] = p_eff
    nl_ref[0] = snl


def _count_body(mid_ref, bits_ref, cnt_ref):
    mid = mid_ref[0]
    le = bits_ref[0] <= mid
    cpart = jnp.sum(le.astype(jnp.int32), axis=0)
    first = pl.program_id(0) == 0

    @pl.when(first)
    def _init():
        cnt_ref[...] = cpart[None, :]

    @pl.when(jnp.logical_not(first))
    def _acc():
        cnt_ref[...] = cnt_ref[...] + cpart[None, :]


def _final_body(thr_ref, p_ref, nl_ref, cnt_ref, sum_ref):
    thr = thr_ref[0]
    p = p_ref[0]
    v = nl_ref[0]
    kept = (p <= thr) & (v >= -0.5)
    cpart = jnp.sum(kept.astype(jnp.float32), axis=0)
    spart = jnp.sum(jnp.where(kept, v, 0.0), axis=0)
    first = pl.program_id(0) == 0

    @pl.when(first)
    def _init():
        cnt_ref[...] = cpart[None, :]
        sum_ref[...] = spart[None, :]

    @pl.when(jnp.logical_not(first))
    def _acc():
        cnt_ref[...] = cnt_ref[...] + cpart[None, :]
        sum_ref[...] = sum_ref[...] + spart[None, :]


_main_call = pl.pallas_call(
    _main_body,
    grid=(_B, _NR),
    in_specs=[
        pl.BlockSpec((1, _C, _RB, _W), lambda i, j: (i, 0, j, 0)),
        pl.BlockSpec((1, _RB, _W), lambda i, j: (i, j, 0)),
    ],
    out_specs=[
        pl.BlockSpec((1, _W), lambda i, j: (0, 0)),
        pl.BlockSpec((1, _W), lambda i, j: (0, 0)),
    ],
    out_shape=[
        jax.ShapeDtypeStruct((1, _W), jnp.float32),
        jax.ShapeDtypeStruct((1, _W), jnp.float32),
    ],
)

_mat_call = pl.pallas_call(
    _mat_body,
    grid=(_B, _NR),
    in_specs=[
        pl.BlockSpec((1, _C, _RB, _W), lambda i, j: (i, 0, j, 0)),
        pl.BlockSpec((1, _RB, _W), lambda i, j: (i, j, 0)),
    ],
    out_specs=[
        pl.BlockSpec((1, _RB, _W), lambda i, j: (i, j, 0)),
        pl.BlockSpec((1, _RB, _W), lambda i, j: (i, j, 0)),
    ],
    out_shape=[
        jax.ShapeDtypeStruct((_B, _H, _W), jnp.float32),
        jax.ShapeDtypeStruct((_B, _H, _W), jnp.float32),
    ],
)

_count_call = pl.pallas_call(
    _count_body,
    grid=(_B,),
    in_specs=[
        pl.BlockSpec(memory_space=pltpu.SMEM),
        pl.BlockSpec((1, _H, _W), lambda i: (i, 0, 0)),
    ],
    out_specs=[pl.BlockSpec((1, _W), lambda i: (0, 0))],
    out_shape=[jax.ShapeDtypeStruct((1, _W), jnp.int32)],
)

_final_call = pl.pallas_call(
    _final_body,
    grid=(_B,),
    in_specs=[
        pl.BlockSpec(memory_space=pltpu.SMEM),
        pl.BlockSpec((1, _H, _W), lambda i: (i, 0, 0)),
        pl.BlockSpec((1, _H, _W), lambda i: (i, 0, 0)),
    ],
    out_specs=[
        pl.BlockSpec((1, _W), lambda i: (0, 0)),
        pl.BlockSpec((1, _W), lambda i: (0, 0)),
    ],
    out_shape=[
        jax.ShapeDtypeStruct((1, _W), jnp.float32),
        jax.ShapeDtypeStruct((1, _W), jnp.float32),
    ],
)


def kernel(pred, target):
    cl, sl = _main_call(pred, target)
    cnt07 = jnp.sum(cl)
    sum07 = jnp.sum(sl)

    def common():
        return sum07 / jnp.maximum(cnt07, 1.0)

    def rare():
        p_eff, snl = _mat_call(pred, target)
        bits = jax.lax.bitcast_convert_type(p_eff, jnp.int32)

        def body(_, lohi):
            lo, hi = lohi
            mid = (lo + hi) // 2
            cnt = jnp.sum(_count_call(mid.reshape(1), bits)[0])
            take_low = cnt >= _K
            return (jnp.where(take_low, lo, mid + 1),
                    jnp.where(take_low, mid, hi))

        lo, _ = jax.lax.fori_loop(
            0, 31, body, (jnp.int32(0), jnp.int32(_TOP_BITS)))
        kth = jax.lax.bitcast_convert_type(lo, jnp.float32)
        thr = jnp.maximum(kth, jnp.float32(_THRESH))

        cf, sf = _final_call(thr.reshape(1), p_eff, snl)
        return jnp.sum(sf) / jnp.maximum(jnp.sum(cf), 1.0)

    return jax.lax.cond(cnt07 >= _K, common, rare)


# speedup vs baseline: 47.6813x; 1.1513x over previous
import functools

import jax
import jax.numpy as jnp
from jax.experimental import pallas as pl
from jax.experimental.pallas import tpu as pltpu
from jax.experimental.pallas import tpu_sc as plsc

_IGNORE = 255
_THRESH = 0.7
_K = 131072

_B, _C, _H, _W = 8, 19, 512, 512
_RB = 128
_NR = _H // _RB

_TOP_BITS = 0x3F800000


def _pixel_stats(pred_ref, tgt_ref):
    x = pred_ref[0]
    t = tgt_ref[0]
    m = jnp.max(x, axis=0)
    xs = x - m[None, :, :]
    e = jnp.exp(xs)
    s = jnp.sum(e, axis=0)
    cls = jax.lax.broadcasted_iota(jnp.int32, (_C, _RB, _W), 0)
    onehot = cls == t[None, :, :]
    shift = jnp.sum(jnp.where(onehot, xs, 0.0), axis=0)
    valid = t != _IGNORE
    p = jnp.exp(shift) / s
    p_eff = jnp.where(valid, p, 1.0)
    nl = jnp.log(s) - shift
    snl = jnp.where(valid, nl, -1.0)
    return p_eff, snl, nl


def _main_body(pred_ref, tgt_ref, cnt_ref, sum_ref):
    p_eff, _, nl = _pixel_stats(pred_ref, tgt_ref)
    kept = p_eff <= _THRESH
    cpart = jnp.sum(kept.astype(jnp.float32), axis=0)
    spart = jnp.sum(jnp.where(kept, nl, 0.0), axis=0)
    first = (pl.program_id(0) == 0) & (pl.program_id(1) == 0)

    @pl.when(first)
    def _init():
        cnt_ref[...] = cpart[None, :]
        sum_ref[...] = spart[None, :]

    @pl.when(jnp.logical_not(first))
    def _acc():
        cnt_ref[...] = cnt_ref[...] + cpart[None, :]
        sum_ref[...] = sum_ref[...] + spart[None, :]


def _mat_body(pred_ref, tgt_ref, p_ref, nl_ref):
    p_eff, snl, _ = _pixel_stats(pred_ref, tgt_ref)
    p_ref[0] = p_eff
    nl_ref[0] = snl


def _count_body(mid_ref, bits_ref, cnt_ref):
    mid = mid_ref[0]
    le = bits_ref[0] <= mid
    cpart = jnp.sum(le.astype(jnp.int32), axis=0)
    first = pl.program_id(0) == 0

    @pl.when(first)
    def _init():
        cnt_ref[...] = cpart[None, :]

    @pl.when(jnp.logical_not(first))
    def _acc():
        cnt_ref[...] = cnt_ref[...] + cpart[None, :]


def _final_body(thr_ref, p_ref, nl_ref, cnt_ref, sum_ref):
    thr = thr_ref[0]
    p = p_ref[0]
    v = nl_ref[0]
    kept = (p <= thr) & (v >= -0.5)
    cpart = jnp.sum(kept.astype(jnp.float32), axis=0)
    spart = jnp.sum(jnp.where(kept, v, 0.0), axis=0)
    first = pl.program_id(0) == 0

    @pl.when(first)
    def _init():
        cnt_ref[...] = cpart[None, :]
        sum_ref[...] = spart[None, :]

    @pl.when(jnp.logical_not(first))
    def _acc():
        cnt_ref[...] = cnt_ref[...] + cpart[None, :]
        sum_ref[...] = sum_ref[...] + spart[None, :]


_main_call = pl.pallas_call(
    _main_body,
    grid=(_B, _NR),
    in_specs=[
        pl.BlockSpec((1, _C, _RB, _W), lambda i, j: (i, 0, j, 0)),
        pl.BlockSpec((1, _RB, _W), lambda i, j: (i, j, 0)),
    ],
    out_specs=[
        pl.BlockSpec((1, _W), lambda i, j: (0, 0)),
        pl.BlockSpec((1, _W), lambda i, j: (0, 0)),
    ],
    out_shape=[
        jax.ShapeDtypeStruct((1, _W), jnp.float32),
        jax.ShapeDtypeStruct((1, _W), jnp.float32),
    ],
)

_mat_call = pl.pallas_call(
    _mat_body,
    grid=(_B, _NR),
    in_specs=[
        pl.BlockSpec((1, _C, _RB, _W), lambda i, j: (i, 0, j, 0)),
        pl.BlockSpec((1, _RB, _W), lambda i, j: (i, j, 0)),
    ],
    out_specs=[
        pl.BlockSpec((1, _RB, _W), lambda i, j: (i, j, 0)),
        pl.BlockSpec((1, _RB, _W), lambda i, j: (i, j, 0)),
    ],
    out_shape=[
        jax.ShapeDtypeStruct((_B, _H, _W), jnp.float32),
        jax.ShapeDtypeStruct((_B, _H, _W), jnp.float32),
    ],
)

_count_call = pl.pallas_call(
    _count_body,
    grid=(_B,),
    in_specs=[
        pl.BlockSpec(memory_space=pltpu.SMEM),
        pl.BlockSpec((1, _H, _W), lambda i: (i, 0, 0)),
    ],
    out_specs=[pl.BlockSpec((1, _W), lambda i: (0, 0))],
    out_shape=[jax.ShapeDtypeStruct((1, _W), jnp.int32)],
)

_final_call = pl.pallas_call(
    _final_body,
    grid=(_B,),
    in_specs=[
        pl.BlockSpec(memory_space=pltpu.SMEM),
        pl.BlockSpec((1, _H, _W), lambda i: (i, 0, 0)),
        pl.BlockSpec((1, _H, _W), lambda i: (i, 0, 0)),
    ],
    out_specs=[
        pl.BlockSpec((1, _W), lambda i: (0, 0)),
        pl.BlockSpec((1, _W), lambda i: (0, 0)),
    ],
    out_shape=[
        jax.ShapeDtypeStruct((1, _W), jnp.float32),
        jax.ShapeDtypeStruct((1, _W), jnp.float32),
    ],
)


def kernel(pred, target):
    cl, sl = _main_call(pred, target)
    cnt07 = jnp.sum(cl)
    sum07 = jnp.sum(sl)

    def common():
        return sum07 / jnp.maximum(cnt07, 1.0)

    def rare():
        p_eff, snl = _mat_call(pred, target)
        bits = jax.lax.bitcast_convert_type(p_eff, jnp.int32)

        def body(_, lohi):
            lo, hi = lohi
            mid = (lo + hi) // 2
            cnt = jnp.sum(_count_call(mid.reshape(1), bits)[0])
            take_low = cnt >= _K
            return (jnp.where(take_low, lo, mid + 1),
                    jnp.where(take_low, mid, hi))

        lo, _ = jax.lax.fori_loop(
            0, 31, body, (jnp.int32(0), jnp.int32(_TOP_BITS)))
        kth = jax.lax.bitcast_convert_type(lo, jnp.float32)
        thr = jnp.maximum(kth, jnp.float32(_THRESH))

        cf, sf = _final_call(thr.reshape(1), p_eff, snl)
        return jnp.sum(sf) / jnp.maximum(jnp.sum(cf), 1.0)

    return jax.lax.cond(cnt07 >= _K, common, rare)
